# Initial kernel scaffold; baseline (speedup 1.0000x reference)
import jax
import jax.numpy as jnp
from jax.experimental import pallas as pl


def kernel(output, target, mask):
    raise NotImplementedError("write your pallas kernel here")


# speedup vs baseline: 1.1135x; 1.1135x over previous
import functools

import jax
import jax.numpy as jnp
from jax import lax
from jax.experimental import pallas as pl
from jax.experimental.pallas import tpu as pltpu
from jax.experimental.pallas import tpu_sc as plsc

B = 8
IMG = 512 * 512
N = B * IMG
NC = 2
NS = 16
NW = NC * NS
PER_TILE = N // NW
TILES_PER_B = NW // B
LANES = 16
CHUNK = 16384
NCHUNK = PER_TILE // CHUNK
VPC = CHUNK // LANES

_mesh = plsc.VectorSubcoreMesh(core_axis_name="c", subcore_axis_name="s")


@functools.partial(
    pl.kernel,
    mesh=_mesh,
    out_type=jax.ShapeDtypeStruct((NW, 8, LANES), jnp.float32),
    scratch_types=[
        pltpu.VMEM((CHUNK,), jnp.float32),
        pltpu.VMEM((CHUNK,), jnp.float32),
        pltpu.VMEM((CHUNK,), jnp.int32),
        pltpu.VMEM((8, LANES), jnp.float32),
    ],
)
def _partial_sums(out_hbm, tgt_hbm, msk_hbm, part_hbm, obuf, tbuf, mbuf, pvec):
    wid = lax.axis_index("s") * NC + lax.axis_index("c")
    base = wid * PER_TILE
    zero = jnp.zeros((LANES,), jnp.float32)
    one = jnp.ones((LANES,), jnp.float32)

    def chunk_body(ci, carry):
        off = base + ci * CHUNK
        pltpu.sync_copy(out_hbm.at[pl.ds(off, CHUNK)], obuf)
        pltpu.sync_copy(tgt_hbm.at[pl.ds(off, CHUNK)], tbuf)
        pltpu.sync_copy(msk_hbm.at[pl.ds(off, CHUNK)], mbuf)

        def vec_body(j, acc):
            a1, a2, a3, a4, c1, c2, c3, c4 = acc
            sl = pl.ds(j * LANES, LANES)
            o = obuf[sl]
            t = tbuf[sl]
            m = mbuf[sl]
            d = o - t
            d2 = d * d
            s1 = m == 1
            s2 = m == 2
            s3 = m == 3
            s4 = m == 4
            a1 = a1 + jnp.where(s1, d2, zero)
            a2 = a2 + jnp.where(s2, d2, zero)
            a3 = a3 + jnp.where(s3, d2, zero)
            a4 = a4 + jnp.where(s4, d2, zero)
            c1 = c1 + jnp.where(s1, one, zero)
            c2 = c2 + jnp.where(s2, one, zero)
            c3 = c3 + jnp.where(s3, one, zero)
            c4 = c4 + jnp.where(s4, one, zero)
            return (a1, a2, a3, a4, c1, c2, c3, c4)

        return lax.fori_loop(0, VPC, vec_body, carry)

    acc = lax.fori_loop(0, NCHUNK, chunk_body, (zero,) * 8)
    for k in range(8):
        pvec[k, :] = acc[k]
    pltpu.sync_copy(pvec, part_hbm.at[wid])


@functools.partial(
    pl.kernel,
    mesh=_mesh,
    out_type=jax.ShapeDtypeStruct((LANES,), jnp.float32),
    scratch_types=[
        pltpu.VMEM((NW, 8, LANES), jnp.float32),
        pltpu.VMEM((LANES,), jnp.float32),
    ],
)
def _combine(part_hbm, out_hbm, pbuf, obuf):
    wid = lax.axis_index("s") * NC + lax.axis_index("c")

    @pl.when(wid == 0)
    def _():
        pltpu.sync_copy(part_hbm, pbuf)
        zero = jnp.zeros((LANES,), jnp.float32)
        lanes = lax.iota(jnp.int32, LANES)
        perms = [jnp.reshape(jnp.bitwise_xor(lanes, d), (LANES, 1))
                 for d in (1, 2, 4, 8)]
        dn = lax.GatherDimensionNumbers(
            offset_dims=(), collapsed_slice_dims=(0,), start_index_map=(0,))

        def lane_sum(v):
            for p in perms:
                v = v + lax.gather(v, p, dn, slice_sizes=(1,),
                                   mode=lax.GatherScatterMode.PROMISE_IN_BOUNDS)
            return v

        lossv = zero
        for b in range(B):
            t0 = TILES_PER_B * b
            for i in range(4):
                v = (pbuf[t0 + 0, i, :] + pbuf[t0 + 1, i, :]
                     + pbuf[t0 + 2, i, :] + pbuf[t0 + 3, i, :])
                c = (pbuf[t0 + 0, i + 4, :] + pbuf[t0 + 1, i + 4, :]
                     + pbuf[t0 + 2, i + 4, :] + pbuf[t0 + 3, i + 4, :])
                sv = lane_sum(v)
                cv = lane_sum(c)
                contrib = jnp.where(cv > 0.0, sv / jnp.maximum(cv, 1.0), zero)
                lossv = lossv + contrib
        obuf[...] = lossv * jnp.float32(1.0 / B)
        pltpu.sync_copy(obuf, out_hbm)


def kernel(output, target, mask):
    o = output.reshape(N)
    t = target.reshape(N)
    m = mask.reshape(N)
    part = _partial_sums(o, t, m)
    res = _combine(part)
    return res[0]
